# baseline (speedup 1.0000x reference)
import dataclasses
import functools

import jax
import jax.numpy as jnp
from jax import lax
from jax.experimental import pallas as pl
from jax.experimental.pallas import tpu as pltpu
from jax.experimental.pallas import tpu_sc as plsc

N = 100000
D = 256
W = 160
NCHUNK = N // W
NW = 32
KMAX = -(-NCHUNK // NW)
L = 16


def _sc_gather(table, xflat):
    mesh = plsc.VectorSubcoreMesh(core_axis_name="c", subcore_axis_name="s")
    cp = pltpu.CompilerParams()
    if "needs_layout_passes" in pltpu.CompilerParams.__dataclass_fields__:
        cp = dataclasses.replace(cp, needs_layout_passes=False)

    @functools.partial(
        pl.kernel,
        mesh=mesh,
        compiler_params=cp,
        out_type=jax.ShapeDtypeStruct((N, D), jnp.float32),
        scratch_types=[
            pltpu.VMEM((9, D), jnp.float32),
            pltpu.VMEM((2 * W,), jnp.int32),
            pltpu.VMEM((2, W // 2), jnp.int32),
            pltpu.VMEM((W, D), jnp.float32),
            pltpu.VMEM((W, D), jnp.float32),
            pltpu.SemaphoreType.DMA,
            pltpu.SemaphoreType.DMA,
            pltpu.SemaphoreType.DMA,
        ],
    )
    def k(table_hbm, xflat_hbm, out_hbm, tab_v, xv, idxv, rows0, rows1,
          gsem, wsem0, wsem1):
        wid = lax.axis_index("s") * 2 + lax.axis_index("c")
        rows = (rows0, rows1)
        wsem = (wsem0, wsem1)
        iota = lax.iota(jnp.int32, L)

        def chunk_of(kk):
            c = kk * NW + wid
            return jnp.where(c < NCHUNK, c, c - NCHUNK)

        def fetch_and_gather(chunk, buf):
            pltpu.sync_copy(xflat_hbm.at[pl.ds(chunk * 2 * W, 2 * W)], xv)
            for g in range(W // L):
                ev = plsc.load_gather(xv, [iota * 2 + (2 * L * g)])
                od = plsc.load_gather(xv, [iota * 2 + (2 * L * g + 1)])
                idxv[g // 5, pl.ds((g % 5) * L, L)] = ev * 3 + od
            c0 = pltpu.async_copy(
                table_hbm.at[idxv.at[0]], buf.at[pl.ds(0, W // 2)], gsem)
            c1 = pltpu.async_copy(
                table_hbm.at[idxv.at[1]], buf.at[pl.ds(W // 2, W // 2)], gsem)
            c0.wait()
            c1.wait()

        def start_write(chunk, b):
            pltpu.async_copy(rows[b], out_hbm.at[pl.ds(chunk * W, W)], wsem[b])

        def wait_write(b):
            pltpu.make_async_copy(
                rows[b], out_hbm.at[pl.ds(0, W)], wsem[b]).wait()

        for kk in range(2):
            c = chunk_of(kk)
            fetch_and_gather(c, rows[kk])
            start_write(c, kk)

        @pl.loop(1, KMAX // 2)
        def _(i):
            for b in range(2):
                kk = i * 2 + b
                wait_write(b)
                c = chunk_of(kk)
                fetch_and_gather(c, rows[b])
                start_write(c, b)

        wait_write(0)
        wait_write(1)

    return k(table, xflat)


def kernel(x, node_type_table, num_inv_pred_table):
    combined = jnp.concatenate(
        [jnp.repeat(node_type_table, 3, axis=0),
         jnp.tile(num_inv_pred_table, (3, 1))],
        axis=1,
    )
    xflat = x.astype(jnp.int32).reshape(-1)
    return _sc_gather(combined, xflat)


# speedup vs baseline: 1.9154x; 1.9154x over previous
import dataclasses
import functools

import jax
import jax.numpy as jnp
from jax import lax
from jax.experimental import pallas as pl
from jax.experimental.pallas import tpu as pltpu
from jax.experimental.pallas import tpu_sc as plsc

N = 100000
D = 256
P = N // 2
W = 80
NCHUNK = P // W
NW = 32
KMAX = -(-NCHUNK // NW)
L = 16


def _sc_gather(table, xflat):
    mesh = plsc.VectorSubcoreMesh(core_axis_name="c", subcore_axis_name="s")
    cp = pltpu.CompilerParams()
    if "needs_layout_passes" in pltpu.CompilerParams.__dataclass_fields__:
        cp = dataclasses.replace(cp, needs_layout_passes=False)

    @functools.partial(
        pl.kernel,
        mesh=mesh,
        compiler_params=cp,
        out_type=jax.ShapeDtypeStruct((P, 2 * D), jnp.float32),
        scratch_types=[
            pltpu.VMEM((4 * W,), jnp.int32),
            pltpu.VMEM((1, W), jnp.int32),
            pltpu.VMEM((W, 2 * D), jnp.float32),
            pltpu.VMEM((W, 2 * D), jnp.float32),
            pltpu.SemaphoreType.DMA,
            pltpu.SemaphoreType.DMA,
            pltpu.SemaphoreType.DMA,
        ],
    )
    def k(table_hbm, xflat_hbm, out_hbm, xv, idxv, rows0, rows1,
          gsem, wsem0, wsem1):
        wid = lax.axis_index("s") * 2 + lax.axis_index("c")
        rows = (rows0, rows1)
        wsem = (wsem0, wsem1)
        iota = lax.iota(jnp.int32, L)

        def chunk_of(kk):
            c = kk * NW + wid
            return jnp.where(c < NCHUNK, c, c - NCHUNK)

        def fetch_and_gather(chunk, buf):
            pltpu.sync_copy(xflat_hbm.at[pl.ds(chunk * 4 * W, 4 * W)], xv)
            for g in range(W // L):
                a = plsc.load_gather(xv, [iota * 4 + (4 * L * g)])
                b = plsc.load_gather(xv, [iota * 4 + (4 * L * g + 1)])
                c = plsc.load_gather(xv, [iota * 4 + (4 * L * g + 2)])
                d = plsc.load_gather(xv, [iota * 4 + (4 * L * g + 3)])
                idxv[0, pl.ds(g * L, L)] = a * 27 + b * 9 + c * 3 + d
            pltpu.async_copy(table_hbm.at[idxv.at[0]], buf, gsem).wait()

        def start_write(chunk, b):
            pltpu.async_copy(rows[b], out_hbm.at[pl.ds(chunk * W, W)], wsem[b])

        def wait_write(b):
            pltpu.make_async_copy(
                rows[b], out_hbm.at[pl.ds(0, W)], wsem[b]).wait()

        for kk in range(2):
            c = chunk_of(kk)
            fetch_and_gather(c, rows[kk])
            start_write(c, kk)

        @pl.loop(1, KMAX // 2)
        def _(i):
            for b in range(2):
                kk = i * 2 + b
                wait_write(b)
                c = chunk_of(kk)
                fetch_and_gather(c, rows[b])
                start_write(c, b)

        wait_write(0)
        wait_write(1)

    return k(table, xflat)


def kernel(x, node_type_table, num_inv_pred_table):
    combined = jnp.concatenate(
        [jnp.repeat(node_type_table, 3, axis=0),
         jnp.tile(num_inv_pred_table, (3, 1))],
        axis=1,
    )
    pair = jnp.concatenate(
        [jnp.repeat(combined, 9, axis=0),
         jnp.tile(combined, (9, 1))],
        axis=1,
    )
    xflat = x.astype(jnp.int32).reshape(-1)
    return _sc_gather(pair, xflat).reshape(N, D)


# speedup vs baseline: 2.1424x; 1.1185x over previous
import dataclasses
import functools

import jax
import jax.numpy as jnp
from jax import lax
from jax.experimental import pallas as pl
from jax.experimental.pallas import tpu as pltpu
from jax.experimental.pallas import tpu_sc as plsc

N = 100000
D = 256
Q = N // 4
W = 40
NCHUNK = Q // W
NW = 32
KMAX = -(-NCHUNK // NW)
L = 16
WPAD = 48


def _sc_gather(table, xflat):
    mesh = plsc.VectorSubcoreMesh(core_axis_name="c", subcore_axis_name="s")
    cp = pltpu.CompilerParams()
    if "needs_layout_passes" in pltpu.CompilerParams.__dataclass_fields__:
        cp = dataclasses.replace(cp, needs_layout_passes=False)

    @functools.partial(
        pl.kernel,
        mesh=mesh,
        compiler_params=cp,
        out_type=jax.ShapeDtypeStruct((Q, 4 * D), jnp.float32),
        scratch_types=[
            pltpu.VMEM((8 * WPAD,), jnp.int32),
            pltpu.VMEM((WPAD,), jnp.int32),
            pltpu.VMEM((W, 4 * D), jnp.float32),
            pltpu.VMEM((W, 4 * D), jnp.float32),
            pltpu.SemaphoreType.DMA,
            pltpu.SemaphoreType.DMA,
            pltpu.SemaphoreType.DMA,
        ],
    )
    def k(table_hbm, xflat_hbm, out_hbm, xv, idxv, rows0, rows1,
          gsem, wsem0, wsem1):
        wid = lax.axis_index("s") * 2 + lax.axis_index("c")
        rows = (rows0, rows1)
        wsem = (wsem0, wsem1)
        iota = lax.iota(jnp.int32, L)

        def chunk_of(kk):
            c = kk * NW + wid
            return jnp.where(c < NCHUNK, c, c - NCHUNK)

        def fetch_and_gather(chunk, buf):
            pltpu.sync_copy(xflat_hbm.at[pl.ds(chunk * 8 * W, 8 * W)],
                            xv.at[pl.ds(0, 8 * W)])
            for g in range(WPAD // L):
                v = plsc.load_gather(xv, [iota * 8 + (8 * L * g)])
                for i in range(1, 8):
                    a = plsc.load_gather(xv, [iota * 8 + (8 * L * g + i)])
                    v = v * 3 + a
                idxv[pl.ds(g * L, L)] = v
            pltpu.async_copy(
                table_hbm.at[idxv.at[pl.ds(0, W)]], buf, gsem).wait()

        def start_write(chunk, b):
            pltpu.async_copy(rows[b], out_hbm.at[pl.ds(chunk * W, W)], wsem[b])

        def wait_write(b):
            pltpu.make_async_copy(
                rows[b], out_hbm.at[pl.ds(0, W)], wsem[b]).wait()

        for kk in range(2):
            c = chunk_of(kk)
            fetch_and_gather(c, rows[kk])
            start_write(c, kk)

        @pl.loop(1, KMAX // 2)
        def _(i):
            for b in range(2):
                kk = i * 2 + b
                wait_write(b)
                c = chunk_of(kk)
                fetch_and_gather(c, rows[b])
                start_write(c, b)

        wait_write(0)
        wait_write(1)

    return k(table, xflat)


def kernel(x, node_type_table, num_inv_pred_table):
    c9 = jnp.concatenate(
        [jnp.repeat(node_type_table, 3, axis=0),
         jnp.tile(num_inv_pred_table, (3, 1))],
        axis=1,
    )
    c81 = jnp.concatenate(
        [jnp.repeat(c9, 9, axis=0), jnp.tile(c9, (9, 1))], axis=1)
    c6561 = jnp.concatenate(
        [jnp.repeat(c81, 81, axis=0), jnp.tile(c81, (81, 1))], axis=1)
    xflat = x.astype(jnp.int32).reshape(-1)
    return _sc_gather(c6561, xflat).reshape(N, D)


# speedup vs baseline: 2.1788x; 1.0170x over previous
import dataclasses
import functools

import jax
import jax.numpy as jnp
from jax import lax
from jax.experimental import pallas as pl
from jax.experimental.pallas import tpu as pltpu
from jax.experimental.pallas import tpu_sc as plsc

N = 100000
D = 256
Q = N // 4
W = 40
NCHUNK = Q // W
NW = 32
KMAX = -(-NCHUNK // NW)
L = 16
WPAD = 48


def _sc_gather(table, xflat):
    mesh = plsc.VectorSubcoreMesh(core_axis_name="c", subcore_axis_name="s")
    cp = pltpu.CompilerParams()
    if "needs_layout_passes" in pltpu.CompilerParams.__dataclass_fields__:
        cp = dataclasses.replace(cp, needs_layout_passes=False)

    @functools.partial(
        pl.kernel,
        mesh=mesh,
        compiler_params=cp,
        out_type=jax.ShapeDtypeStruct((Q, 4 * D), jnp.float32),
        scratch_types=[
            pltpu.VMEM((8 * WPAD,), jnp.int32),
            pltpu.VMEM((WPAD,), jnp.int32),
            pltpu.VMEM((W, 4 * D), jnp.float32),
            pltpu.VMEM((W, 4 * D), jnp.float32),
            pltpu.SemaphoreType.DMA,
            pltpu.SemaphoreType.DMA,
            pltpu.SemaphoreType.DMA,
        ],
    )
    def k(table_hbm, xflat_hbm, out_hbm, xv, idxv, rows0, rows1,
          gsem, wsem0, wsem1):
        wid = lax.axis_index("s") * 2 + lax.axis_index("c")
        rows = (rows0, rows1)
        wsem = (wsem0, wsem1)
        iota = lax.iota(jnp.int32, L)

        def chunk_of(kk):
            c = kk * NW + wid
            return jnp.where(c < NCHUNK, c, c - NCHUNK)

        def fetch_and_gather(chunk, buf):
            pltpu.sync_copy(xflat_hbm.at[pl.ds(chunk * 8 * W, 8 * W)],
                            xv.at[pl.ds(0, 8 * W)])
            for g in range(WPAD // L):
                v = plsc.load_gather(xv, [iota * 8 + (8 * L * g)])
                for i in range(1, 8):
                    a = plsc.load_gather(xv, [iota * 8 + (8 * L * g + i)])
                    v = v * 3 + a
                idxv[pl.ds(g * L, L)] = v
            pltpu.async_copy(
                table_hbm.at[idxv.at[pl.ds(0, W)]], buf, gsem).wait()

        def start_write(chunk, b):
            pltpu.async_copy(rows[b], out_hbm.at[pl.ds(chunk * W, W)], wsem[b])

        def wait_write(b):
            pltpu.make_async_copy(
                rows[b], out_hbm.at[pl.ds(0, W)], wsem[b]).wait()

        for kk in range(2):
            c = chunk_of(kk)
            fetch_and_gather(c, rows[kk])
            start_write(c, kk)

        @pl.loop(1, KMAX // 2)
        def _(i):
            for b in range(2):
                kk = i * 2 + b
                wait_write(b)
                c = chunk_of(kk)
                fetch_and_gather(c, rows[b])
                start_write(c, b)

        wait_write(0)
        wait_write(1)

    return k(table, xflat)


def kernel(x, node_type_table, num_inv_pred_table):
    def cross(a, b):
        n, m = a.shape[0], b.shape[0]
        left = jnp.broadcast_to(a[:, None, :], (n, m, a.shape[1]))
        right = jnp.broadcast_to(b[None, :, :], (n, m, b.shape[1]))
        return jnp.concatenate([left, right], axis=2).reshape(
            n * m, a.shape[1] + b.shape[1])

    c9 = cross(node_type_table, num_inv_pred_table)
    c81 = cross(c9, c9)
    c6561 = cross(c81, c81)
    xflat = x.astype(jnp.int32).reshape(-1)
    return _sc_gather(c6561, xflat).reshape(N, D)


# speedup vs baseline: 2.1846x; 1.0027x over previous
import dataclasses
import functools

import jax
import jax.numpy as jnp
from jax import lax
from jax.experimental import pallas as pl
from jax.experimental.pallas import tpu as pltpu
from jax.experimental.pallas import tpu_sc as plsc

N = 100000
D = 256
Q = N // 4
W = 40
NCHUNK = Q // W
NW = 32
KMAX = -(-NCHUNK // NW)
L = 16
WPAD = 48


def _sc_gather(table, xflat):
    mesh = plsc.VectorSubcoreMesh(core_axis_name="c", subcore_axis_name="s")
    cp = pltpu.CompilerParams()
    if "needs_layout_passes" in pltpu.CompilerParams.__dataclass_fields__:
        cp = dataclasses.replace(cp, needs_layout_passes=False)

    @functools.partial(
        pl.kernel,
        mesh=mesh,
        compiler_params=cp,
        out_type=jax.ShapeDtypeStruct((Q, 4 * D), jnp.float32),
        scratch_types=[
            pltpu.VMEM((8 * WPAD,), jnp.int32),
            pltpu.VMEM((WPAD,), jnp.int32),
            pltpu.VMEM((W, 4 * D), jnp.float32),
            pltpu.VMEM((W, 4 * D), jnp.float32),
            pltpu.SemaphoreType.DMA,
            pltpu.SemaphoreType.DMA,
            pltpu.SemaphoreType.DMA,
        ],
    )
    def k(table_hbm, xflat_hbm, out_hbm, xv, idxv, rows0, rows1,
          gsem, wsem0, wsem1):
        wid = lax.axis_index("s") * 2 + lax.axis_index("c")
        rows = (rows0, rows1)
        wsem = (wsem0, wsem1)
        iota = lax.iota(jnp.int32, L)

        def chunk_of(kk):
            c = kk * NW + wid
            return jnp.where(c < NCHUNK, c, c - NCHUNK)

        def fetch_and_gather(chunk, buf):
            pltpu.sync_copy(xflat_hbm.at[pl.ds(chunk * 8 * W, 8 * W)],
                            xv.at[pl.ds(0, 8 * W)])
            for g in range(WPAD // L):
                v = plsc.load_gather(xv, [iota * 8 + (8 * L * g)])
                for i in range(1, 8):
                    a = plsc.load_gather(xv, [iota * 8 + (8 * L * g + i)])
                    v = v * 3 + a
                idxv[pl.ds(g * L, L)] = v
            h = 24
            g0 = pltpu.async_copy(
                table_hbm.at[idxv.at[pl.ds(0, h)]], buf.at[pl.ds(0, h)], gsem)
            g1 = pltpu.async_copy(
                table_hbm.at[idxv.at[pl.ds(h, W - h)]],
                buf.at[pl.ds(h, W - h)], gsem)
            g0.wait()
            g1.wait()

        def start_write(chunk, b):
            pltpu.async_copy(rows[b], out_hbm.at[pl.ds(chunk * W, W)], wsem[b])

        def wait_write(b):
            pltpu.make_async_copy(
                rows[b], out_hbm.at[pl.ds(0, W)], wsem[b]).wait()

        for kk in range(2):
            c = chunk_of(kk)
            fetch_and_gather(c, rows[kk])
            start_write(c, kk)

        @pl.loop(1, KMAX // 2)
        def _(i):
            for b in range(2):
                kk = i * 2 + b
                wait_write(b)
                c = chunk_of(kk)
                fetch_and_gather(c, rows[b])
                start_write(c, b)

        wait_write(0)
        wait_write(1)

    return k(table, xflat)


def kernel(x, node_type_table, num_inv_pred_table):
    def cross(a, b):
        n, m = a.shape[0], b.shape[0]
        left = jnp.broadcast_to(a[:, None, :], (n, m, a.shape[1]))
        right = jnp.broadcast_to(b[None, :, :], (n, m, b.shape[1]))
        return jnp.concatenate([left, right], axis=2).reshape(
            n * m, a.shape[1] + b.shape[1])

    c9 = cross(node_type_table, num_inv_pred_table)
    c81 = cross(c9, c9)
    c6561 = cross(c81, c81)
    xflat = x.astype(jnp.int32).reshape(-1)
    return _sc_gather(c6561, xflat).reshape(N, D)
